# Initial kernel scaffold; baseline (speedup 1.0000x reference)
import jax
import jax.numpy as jnp
from jax.experimental import pallas as pl


def kernel(imgs, num_transforms_logits, scale_logits, possible_num_sequential_transforms):
    raise NotImplementedError("write your pallas kernel here")


# speedup vs baseline: 2.8125x; 2.8125x over previous
import numpy as np
import jax
import jax.numpy as jnp
from jax.experimental import pallas as pl

B = 16384
T = 8
NUM_TRANSFORMS = 64
NUM_SCALES = 256

R = 256
GRID = B // R

_U32 = np.uint32
_ROT0 = (13, 15, 26, 6)
_ROT1 = (17, 29, 16, 24)


def _threefry_np(k0, k1, x0, x1):
    np.seterr(over="ignore")
    k0, k1 = _U32(k0), _U32(k1)
    ks = [k0, k1, _U32(k0 ^ k1 ^ _U32(0x1BD11BDA))]
    x0 = (x0 + k0).astype(_U32)
    x1 = (x1 + k1).astype(_U32)
    for i in range(5):
        for r in _ROT0 if i % 2 == 0 else _ROT1:
            x0 = (x0 + x1).astype(_U32)
            x1 = ((x1 << _U32(r)) | (x1 >> _U32(32 - r))).astype(_U32)
            x1 = x1 ^ x0
        x0 = (x0 + ks[(i + 1) % 3]).astype(_U32)
        x1 = (x1 + ks[(i + 2) % 3] + _U32(i + 1)).astype(_U32)
    return x0, x1


def _split_np(key, num):
    a, b = _threefry_np(key[0], key[1], np.zeros(num, _U32), np.arange(num, dtype=_U32))
    return list(zip(a.tolist(), b.tolist()))


_K1, _K2, _K3 = _split_np((0, 42), 3)
_K2B = _split_np(_K2, 2)[1]


def _threefry_bits(key, x1):
    k1 = jnp.uint32(key[1])
    ks = (jnp.uint32(key[0]), k1, jnp.uint32(key[0] ^ key[1] ^ 0x1BD11BDA))
    x0 = jnp.full(x1.shape, key[0], jnp.uint32)
    x1 = x1 + k1
    for i in range(5):
        for r in _ROT0 if i % 2 == 0 else _ROT1:
            x0 = x0 + x1
            x1 = (x1 << _U32(r)) | (x1 >> _U32(32 - r))
            x1 = x1 ^ x0
        x0 = x0 + ks[(i + 1) % 3]
        x1 = x1 + ks[(i + 2) % 3] + jnp.uint32(i + 1)
    return x0 ^ x1


def _gumbel(bits):
    tiny = np.float32(np.finfo(np.float32).tiny)
    float_bits = (bits >> _U32(9)) | _U32(0x3F800000)
    floats = jax.lax.bitcast_convert_type(float_bits, jnp.float32) - jnp.float32(1.0)
    u = jnp.maximum(tiny, floats * (np.float32(1.0) - tiny) + tiny)
    return -jnp.log(-jnp.log(u))


def _sampler_kernel(ntl_ref, sl_ref, poss_ref, aug_ref, sc_ref, lp_ref):
    i = pl.program_id(0)

    t_iota = jax.lax.broadcasted_iota(jnp.int32, (R, T), 1)

    base_a = jnp.uint32(R * T) * i.astype(jnp.uint32)
    x1_a = base_a + jax.lax.broadcasted_iota(jnp.uint32, (R, T), 0) * _U32(T) \
        + jax.lax.broadcasted_iota(jnp.uint32, (R, T), 1)
    z_a = _gumbel(_threefry_bits(_K1, x1_a)) + ntl_ref[0:1, :]
    max_a = jnp.max(z_a, axis=1, keepdims=True)
    idx_a = jnp.min(jnp.where(z_a == max_a, t_iota, T), axis=1, keepdims=True)
    num_t = jnp.sum(jnp.where(t_iota == idx_a, poss_ref[0:1, :], 0),
                    axis=1, keepdims=True)
    mask = t_iota >= num_t

    bits_b = _threefry_bits(_K2B, x1_a)
    aug = jnp.where(mask, 0, (bits_b & _U32(63)).astype(jnp.int32))
    aug_ref[...] = aug.reshape(1, R, T)

    base_d = jnp.uint32(R * T * NUM_SCALES) * i.astype(jnp.uint32)
    x1_d = base_d \
        + jax.lax.broadcasted_iota(jnp.uint32, (R, T * NUM_SCALES), 0) * _U32(T * NUM_SCALES) \
        + jax.lax.broadcasted_iota(jnp.uint32, (R, T * NUM_SCALES), 1)
    g_d = _gumbel(_threefry_bits(_K3, x1_d))

    sl = sl_ref[...]
    m_tab = jnp.max(sl, axis=1, keepdims=True)
    lse_tab = jnp.log(jnp.sum(jnp.exp(sl - m_tab), axis=1, keepdims=True)) + m_tab

    ntl = ntl_ref[0:1, :]
    m_nt = jnp.max(ntl, axis=1, keepdims=True)
    lse_nt = jnp.log(jnp.sum(jnp.exp(ntl - m_nt), axis=1, keepdims=True)) + m_nt
    lp_nt = ntl - lse_nt
    lp_sum = jnp.sum(jnp.where(t_iota == idx_a, lp_nt, 0.0),
                     axis=1, keepdims=True)

    j_iota = jax.lax.broadcasted_iota(jnp.int32, (R, NUM_TRANSFORMS), 1)
    s_iota = jax.lax.broadcasted_iota(jnp.int32, (R, NUM_SCALES), 1)
    chosen_cols = []
    for t in range(T):
        onehot = (j_iota == aug[:, t:t + 1]).astype(jnp.float32)
        gathered = jnp.dot(onehot, sl, preferred_element_type=jnp.float32)
        z_t = g_d[:, t * NUM_SCALES:(t + 1) * NUM_SCALES] + gathered
        max_t = jnp.max(z_t, axis=1, keepdims=True)
        chosen = jnp.min(jnp.where(z_t == max_t, s_iota, NUM_SCALES),
                         axis=1, keepdims=True)
        chosen_cols.append(chosen)
        v_sel = jnp.sum(jnp.where(s_iota == chosen, gathered, 0.0),
                        axis=1, keepdims=True)
        lse_g = jnp.dot(onehot, lse_tab, preferred_element_type=jnp.float32)
        lp_sum = lp_sum + jnp.where(mask[:, t:t + 1], 0.0, v_sel - lse_g)

    sc_ref[...] = jnp.concatenate(chosen_cols, axis=1).reshape(1, R, T)
    lp_ref[...] = lp_sum.reshape(1, R, 1)


def kernel(imgs, num_transforms_logits, scale_logits,
           possible_num_sequential_transforms):
    del imgs
    ntl = num_transforms_logits.reshape(1, T)
    poss = possible_num_sequential_transforms.reshape(1, T)

    aug, sc, lp = pl.pallas_call(
        _sampler_kernel,
        grid=(GRID,),
        in_specs=[
            pl.BlockSpec((1, T), lambda i: (0, 0)),
            pl.BlockSpec((NUM_TRANSFORMS, NUM_SCALES), lambda i: (0, 0)),
            pl.BlockSpec((1, T), lambda i: (0, 0)),
        ],
        out_specs=[
            pl.BlockSpec((1, R, T), lambda i: (i, 0, 0)),
            pl.BlockSpec((1, R, T), lambda i: (i, 0, 0)),
            pl.BlockSpec((1, R, 1), lambda i: (i, 0, 0)),
        ],
        out_shape=[
            jax.ShapeDtypeStruct((GRID, R, T), jnp.int32),
            jax.ShapeDtypeStruct((GRID, R, T), jnp.int32),
            jax.ShapeDtypeStruct((GRID, R, 1), jnp.float32),
        ],
    )(ntl, scale_logits, poss)

    return (aug.reshape(B, T), sc.reshape(B, T), lp.reshape(B))


# speedup vs baseline: 2.8150x; 1.0009x over previous
import numpy as np
import jax
import jax.numpy as jnp
from jax.experimental import pallas as pl

B = 16384
T = 8
NUM_TRANSFORMS = 64
NUM_SCALES = 256

R = 256
GRID = B // R

_U32 = np.uint32
_ROT0 = (13, 15, 26, 6)
_ROT1 = (17, 29, 16, 24)


def _threefry_np(k0, k1, x0, x1):
    np.seterr(over="ignore")
    k0, k1 = _U32(k0), _U32(k1)
    ks = [k0, k1, _U32(k0 ^ k1 ^ _U32(0x1BD11BDA))]
    x0 = (x0 + k0).astype(_U32)
    x1 = (x1 + k1).astype(_U32)
    for i in range(5):
        for r in _ROT0 if i % 2 == 0 else _ROT1:
            x0 = (x0 + x1).astype(_U32)
            x1 = ((x1 << _U32(r)) | (x1 >> _U32(32 - r))).astype(_U32)
            x1 = x1 ^ x0
        x0 = (x0 + ks[(i + 1) % 3]).astype(_U32)
        x1 = (x1 + ks[(i + 2) % 3] + _U32(i + 1)).astype(_U32)
    return x0, x1


def _split_np(key, num):
    a, b = _threefry_np(key[0], key[1], np.zeros(num, _U32), np.arange(num, dtype=_U32))
    return list(zip(a.tolist(), b.tolist()))


_K1, _K2, _K3 = _split_np((0, 42), 3)
_K2B = _split_np(_K2, 2)[1]


def _threefry_bits(key, x1, key_prefolded=False):
    ks = (jnp.uint32(key[0]), jnp.uint32(key[1]),
          jnp.uint32(key[0] ^ key[1] ^ 0x1BD11BDA))
    x0 = jnp.full(x1.shape, key[0], jnp.uint32)
    if not key_prefolded:
        x1 = x1 + ks[1]
    for i in range(5):
        for r in _ROT0 if i % 2 == 0 else _ROT1:
            x0 = x0 + x1
            x1 = (x1 << _U32(r)) | (x1 >> _U32(32 - r))
            x1 = x1 ^ x0
        x0 = x0 + ks[(i + 1) % 3]
        x1 = x1 + ks[(i + 2) % 3] + jnp.uint32(i + 1)
    return x0 ^ x1


def _gumbel(bits):
    tiny = np.float32(np.finfo(np.float32).tiny)
    float_bits = (bits >> _U32(9)) | _U32(0x3F800000)
    floats = jax.lax.bitcast_convert_type(float_bits, jnp.float32) - jnp.float32(1.0)
    u = jnp.maximum(tiny, floats * (np.float32(1.0) - tiny) + tiny)
    return -jnp.log(-jnp.log(u))


def _sampler_kernel(iota_a_ref, iota_d_ref, ntl_ref, sl_ref, poss_ref,
                    aug_ref, sc_ref, lp_ref):
    i = pl.program_id(0)

    t_iota = jax.lax.broadcasted_iota(jnp.int32, (R, T), 1)

    base_a = jnp.uint32(R * T) * i.astype(jnp.uint32)
    x1_a = base_a + iota_a_ref[...]
    z_a = _gumbel(_threefry_bits(_K1, x1_a)) + ntl_ref[0:1, :]
    max_a = jnp.max(z_a, axis=1, keepdims=True)
    idx_a = jnp.min(jnp.where(z_a == max_a, t_iota, T), axis=1, keepdims=True)
    num_t = jnp.sum(jnp.where(t_iota == idx_a, poss_ref[0:1, :], 0),
                    axis=1, keepdims=True)
    mask = t_iota >= num_t

    bits_b = _threefry_bits(_K2B, x1_a)
    aug = jnp.where(mask, 0, (bits_b & _U32(63)).astype(jnp.int32))
    aug_ref[...] = aug.reshape(1, R, T)

    base_d = jnp.uint32(R * T * NUM_SCALES) * i.astype(jnp.uint32)
    g_d = _gumbel(_threefry_bits(_K3, base_d + iota_d_ref[...],
                                 key_prefolded=True))

    sl = sl_ref[...]
    m_tab = jnp.max(sl, axis=1, keepdims=True)
    lse_tab = jnp.log(jnp.sum(jnp.exp(sl - m_tab), axis=1, keepdims=True)) + m_tab

    ntl = ntl_ref[0:1, :]
    m_nt = jnp.max(ntl, axis=1, keepdims=True)
    lse_nt = jnp.log(jnp.sum(jnp.exp(ntl - m_nt), axis=1, keepdims=True)) + m_nt
    lp_nt = ntl - lse_nt
    lp_sum = jnp.sum(jnp.where(t_iota == idx_a, lp_nt, 0.0),
                     axis=1, keepdims=True)

    j_iota = jax.lax.broadcasted_iota(jnp.int32, (R, NUM_TRANSFORMS), 1)
    s_iota = jax.lax.broadcasted_iota(jnp.int32, (R, NUM_SCALES), 1)
    chosen_cols = []
    for t in range(T):
        onehot = (j_iota == aug[:, t:t + 1]).astype(jnp.float32)
        gathered = jnp.dot(onehot, sl, preferred_element_type=jnp.float32)
        z_t = g_d[:, t * NUM_SCALES:(t + 1) * NUM_SCALES] + gathered
        max_t = jnp.max(z_t, axis=1, keepdims=True)
        chosen = jnp.min(jnp.where(z_t == max_t, s_iota, NUM_SCALES),
                         axis=1, keepdims=True)
        chosen_cols.append(chosen)
        v_sel = jnp.sum(jnp.where(s_iota == chosen, gathered, 0.0),
                        axis=1, keepdims=True)
        lse_g = jnp.dot(onehot, lse_tab, preferred_element_type=jnp.float32)
        lp_sum = lp_sum + jnp.where(mask[:, t:t + 1], 0.0, v_sel - lse_g)

    sc_ref[...] = jnp.concatenate(chosen_cols, axis=1).reshape(1, R, T)
    lp_ref[...] = lp_sum.reshape(1, R, 1)


def kernel(imgs, num_transforms_logits, scale_logits,
           possible_num_sequential_transforms):
    del imgs
    ntl = num_transforms_logits.reshape(1, T)
    poss = possible_num_sequential_transforms.reshape(1, T)

    row = np.arange(R, dtype=np.uint32)[:, None]
    iota_a = jnp.asarray(row * _U32(T) + np.arange(T, dtype=np.uint32)[None, :])
    iota_d = jnp.asarray(row * _U32(T * NUM_SCALES)
                         + np.arange(T * NUM_SCALES, dtype=np.uint32)[None, :]
                         + _U32(_K3[1]))

    aug, sc, lp = pl.pallas_call(
        _sampler_kernel,
        grid=(GRID,),
        in_specs=[
            pl.BlockSpec((R, T), lambda i: (0, 0)),
            pl.BlockSpec((R, T * NUM_SCALES), lambda i: (0, 0)),
            pl.BlockSpec((1, T), lambda i: (0, 0)),
            pl.BlockSpec((NUM_TRANSFORMS, NUM_SCALES), lambda i: (0, 0)),
            pl.BlockSpec((1, T), lambda i: (0, 0)),
        ],
        out_specs=[
            pl.BlockSpec((1, R, T), lambda i: (i, 0, 0)),
            pl.BlockSpec((1, R, T), lambda i: (i, 0, 0)),
            pl.BlockSpec((1, R, 1), lambda i: (i, 0, 0)),
        ],
        out_shape=[
            jax.ShapeDtypeStruct((GRID, R, T), jnp.int32),
            jax.ShapeDtypeStruct((GRID, R, T), jnp.int32),
            jax.ShapeDtypeStruct((GRID, R, 1), jnp.float32),
        ],
    )(iota_a, iota_d, ntl, scale_logits, poss)

    return (aug.reshape(B, T), sc.reshape(B, T), lp.reshape(B))


# speedup vs baseline: 3.0771x; 1.0931x over previous
import numpy as np
import jax
import jax.numpy as jnp
from jax.experimental import pallas as pl

B = 16384
T = 8
NUM_TRANSFORMS = 64
NUM_SCALES = 256

R = 256
GRID = B // R
TS = T * NUM_SCALES

_U32 = np.uint32
_ROT0 = (13, 15, 26, 6)
_ROT1 = (17, 29, 16, 24)


def _threefry_np(k0, k1, x0, x1):
    np.seterr(over="ignore")
    k0, k1 = _U32(k0), _U32(k1)
    ks = [k0, k1, _U32(k0 ^ k1 ^ _U32(0x1BD11BDA))]
    x0 = (x0 + k0).astype(_U32)
    x1 = (x1 + k1).astype(_U32)
    for i in range(5):
        for r in _ROT0 if i % 2 == 0 else _ROT1:
            x0 = (x0 + x1).astype(_U32)
            x1 = ((x1 << _U32(r)) | (x1 >> _U32(32 - r))).astype(_U32)
            x1 = x1 ^ x0
        x0 = (x0 + ks[(i + 1) % 3]).astype(_U32)
        x1 = (x1 + ks[(i + 2) % 3] + _U32(i + 1)).astype(_U32)
    return x0, x1


def _split_np(key, num):
    a, b = _threefry_np(key[0], key[1], np.zeros(num, _U32), np.arange(num, dtype=_U32))
    return list(zip(a.tolist(), b.tolist()))


_K1, _K2, _K3 = _split_np((0, 42), 3)
_K2B = _split_np(_K2, 2)[1]


def _threefry_bits(key, x1, key_prefolded=False):
    ks = (jnp.uint32(key[0]), jnp.uint32(key[1]),
          jnp.uint32(key[0] ^ key[1] ^ 0x1BD11BDA))
    x0 = jnp.full(x1.shape, key[0], jnp.uint32)
    if not key_prefolded:
        x1 = x1 + ks[1]
    for i in range(5):
        for r in _ROT0 if i % 2 == 0 else _ROT1:
            x0 = x0 + x1
            x1 = (x1 << _U32(r)) | (x1 >> _U32(32 - r))
            x1 = x1 ^ x0
        x0 = x0 + ks[(i + 1) % 3]
        x1 = x1 + ks[(i + 2) % 3] + jnp.uint32(i + 1)
    return x0 ^ x1


def _gumbel(bits):
    tiny = np.float32(np.finfo(np.float32).tiny)
    float_bits = (bits >> _U32(9)) | _U32(0x3F800000)
    floats = jax.lax.bitcast_convert_type(float_bits, jnp.float32) - jnp.float32(1.0)
    u = jnp.maximum(tiny, floats * (np.float32(1.0) - tiny) + tiny)
    return -jnp.log(-jnp.log(u))


def _head_kernel(iota_ref, ntl_ref, poss_ref, aug_ref, numt_ref, lp_ref):
    t_iota = jax.lax.broadcasted_iota(jnp.int32, (T, B), 0)

    z_a = _gumbel(_threefry_bits(_K1, iota_ref[...])) + ntl_ref[...]
    max_a = jnp.max(z_a, axis=0, keepdims=True)
    idx_a = jnp.min(jnp.where(z_a == max_a, t_iota, T), axis=0, keepdims=True)
    num_t = jnp.sum(jnp.where(t_iota == idx_a, poss_ref[...], 0),
                    axis=0, keepdims=True)
    mask = t_iota >= num_t

    bits_b = _threefry_bits(_K2B, iota_ref[...])
    aug_ref[...] = jnp.where(mask, 0, (bits_b & _U32(63)).astype(jnp.int32))
    numt_ref[...] = num_t

    ntl = ntl_ref[...]
    m_nt = jnp.max(ntl, axis=0, keepdims=True)
    lse_nt = jnp.log(jnp.sum(jnp.exp(ntl - m_nt), axis=0, keepdims=True)) + m_nt
    lp_nt = ntl - lse_nt
    lp_ref[...] = jnp.sum(jnp.where(t_iota == idx_a, lp_nt, 0.0),
                          axis=0, keepdims=True)


def _scale_kernel(iota_ref, aug_ref, numt_ref, lphead_ref, sl_ref,
                  sc_ref, lp_ref):
    i = pl.program_id(0)

    t_iota = jax.lax.broadcasted_iota(jnp.int32, (R, T), 1)
    mask = t_iota >= numt_ref[...]
    aug = aug_ref[...]

    base_d = jnp.uint32(R * TS) * i.astype(jnp.uint32)
    g_d = _gumbel(_threefry_bits(_K3, base_d + iota_ref[...],
                                 key_prefolded=True))

    sl = sl_ref[...]
    m_tab = jnp.max(sl, axis=1, keepdims=True)
    lse_tab = jnp.log(jnp.sum(jnp.exp(sl - m_tab), axis=1, keepdims=True)) + m_tab

    j_iota = jax.lax.broadcasted_iota(jnp.int32, (R, NUM_TRANSFORMS), 1)
    s_iota = jax.lax.broadcasted_iota(jnp.int32, (R, NUM_SCALES), 1)
    lp_sum = lphead_ref[...]
    chosen_cols = []
    for t in range(T):
        onehot = (j_iota == aug[:, t:t + 1]).astype(jnp.float32)
        gathered = jnp.dot(onehot, sl, preferred_element_type=jnp.float32)
        z_t = g_d[:, t * NUM_SCALES:(t + 1) * NUM_SCALES] + gathered
        max_t = jnp.max(z_t, axis=1, keepdims=True)
        chosen = jnp.min(jnp.where(z_t == max_t, s_iota, NUM_SCALES),
                         axis=1, keepdims=True)
        chosen_cols.append(chosen)
        v_sel = jnp.sum(jnp.where(s_iota == chosen, gathered, 0.0),
                        axis=1, keepdims=True)
        lse_g = jnp.dot(onehot, lse_tab, preferred_element_type=jnp.float32)
        lp_sum = lp_sum + jnp.where(mask[:, t:t + 1], 0.0, v_sel - lse_g)

    sc_ref[...] = jnp.concatenate(chosen_cols, axis=1).reshape(1, R, T)
    lp_ref[...] = lp_sum.reshape(1, R, 1)


def kernel(imgs, num_transforms_logits, scale_logits,
           possible_num_sequential_transforms):
    del imgs
    ntl = num_transforms_logits.reshape(T, 1)
    poss = possible_num_sequential_transforms.reshape(T, 1)

    iota_head = jnp.asarray(np.arange(B, dtype=np.uint32)[None, :] * _U32(T)
                            + np.arange(T, dtype=np.uint32)[:, None])
    iota_d = jnp.asarray(np.arange(R, dtype=np.uint32)[:, None] * _U32(TS)
                         + np.arange(TS, dtype=np.uint32)[None, :]
                         + _U32(_K3[1]))

    aug8, numt, lphead = pl.pallas_call(
        _head_kernel,
        in_specs=[
            pl.BlockSpec((T, B), lambda: (0, 0)),
            pl.BlockSpec((T, 1), lambda: (0, 0)),
            pl.BlockSpec((T, 1), lambda: (0, 0)),
        ],
        out_specs=[
            pl.BlockSpec((T, B), lambda: (0, 0)),
            pl.BlockSpec((1, B), lambda: (0, 0)),
            pl.BlockSpec((1, B), lambda: (0, 0)),
        ],
        out_shape=[
            jax.ShapeDtypeStruct((T, B), jnp.int32),
            jax.ShapeDtypeStruct((1, B), jnp.int32),
            jax.ShapeDtypeStruct((1, B), jnp.float32),
        ],
    )(iota_head, ntl, poss)

    aug = aug8.T
    numt_col = numt.reshape(B, 1)
    lphead_col = lphead.reshape(B, 1)

    sc, lp = pl.pallas_call(
        _scale_kernel,
        grid=(GRID,),
        in_specs=[
            pl.BlockSpec((R, TS), lambda i: (0, 0)),
            pl.BlockSpec((R, T), lambda i: (i, 0)),
            pl.BlockSpec((R, 1), lambda i: (i, 0)),
            pl.BlockSpec((R, 1), lambda i: (i, 0)),
            pl.BlockSpec((NUM_TRANSFORMS, NUM_SCALES), lambda i: (0, 0)),
        ],
        out_specs=[
            pl.BlockSpec((1, R, T), lambda i: (i, 0, 0)),
            pl.BlockSpec((1, R, 1), lambda i: (i, 0, 0)),
        ],
        out_shape=[
            jax.ShapeDtypeStruct((GRID, R, T), jnp.int32),
            jax.ShapeDtypeStruct((GRID, R, 1), jnp.float32),
        ],
    )(iota_d, aug, numt_col, lphead_col, scale_logits)

    return (aug, sc.reshape(B, T), lp.reshape(B))


# speedup vs baseline: 3.2638x; 1.0607x over previous
import numpy as np
import jax
import jax.numpy as jnp
from jax.experimental import pallas as pl

B = 16384
T = 8
NUM_TRANSFORMS = 64
NUM_SCALES = 256

R = 512
GRID = B // R
TS = T * NUM_SCALES

_U32 = np.uint32
_ROT0 = (13, 15, 26, 6)
_ROT1 = (17, 29, 16, 24)


def _threefry_np(k0, k1, x0, x1):
    np.seterr(over="ignore")
    k0, k1 = _U32(k0), _U32(k1)
    ks = [k0, k1, _U32(k0 ^ k1 ^ _U32(0x1BD11BDA))]
    x0 = (x0 + k0).astype(_U32)
    x1 = (x1 + k1).astype(_U32)
    for i in range(5):
        for r in _ROT0 if i % 2 == 0 else _ROT1:
            x0 = (x0 + x1).astype(_U32)
            x1 = ((x1 << _U32(r)) | (x1 >> _U32(32 - r))).astype(_U32)
            x1 = x1 ^ x0
        x0 = (x0 + ks[(i + 1) % 3]).astype(_U32)
        x1 = (x1 + ks[(i + 2) % 3] + _U32(i + 1)).astype(_U32)
    return x0, x1


def _split_np(key, num):
    a, b = _threefry_np(key[0], key[1], np.zeros(num, _U32), np.arange(num, dtype=_U32))
    return list(zip(a.tolist(), b.tolist()))


_K1, _K2, _K3 = _split_np((0, 42), 3)
_K2B = _split_np(_K2, 2)[1]


def _threefry_bits(key, x1, key_prefolded=False):
    ks = (jnp.uint32(key[0]), jnp.uint32(key[1]),
          jnp.uint32(key[0] ^ key[1] ^ 0x1BD11BDA))
    x0 = jnp.full(x1.shape, key[0], jnp.uint32)
    if not key_prefolded:
        x1 = x1 + ks[1]
    for i in range(5):
        for r in _ROT0 if i % 2 == 0 else _ROT1:
            x0 = x0 + x1
            x1 = (x1 << _U32(r)) | (x1 >> _U32(32 - r))
            x1 = x1 ^ x0
        x0 = x0 + ks[(i + 1) % 3]
        x1 = x1 + ks[(i + 2) % 3] + jnp.uint32(i + 1)
    return x0 ^ x1


def _gumbel(bits):
    tiny = np.float32(np.finfo(np.float32).tiny)
    float_bits = (bits >> _U32(9)) | _U32(0x3F800000)
    floats = jax.lax.bitcast_convert_type(float_bits, jnp.float32) - jnp.float32(1.0)
    u = jnp.maximum(tiny, floats * (np.float32(1.0) - tiny) + tiny)
    return -jnp.log(-jnp.log(u))


def _head_kernel(iota_ref, ntl_ref, poss_ref, aug_ref, numt_ref, lp_ref):
    t_iota = jax.lax.broadcasted_iota(jnp.int32, (T, B), 0)

    z_a = _gumbel(_threefry_bits(_K1, iota_ref[...])) + ntl_ref[...]
    max_a = jnp.max(z_a, axis=0, keepdims=True)
    idx_a = jnp.min(jnp.where(z_a == max_a, t_iota, T), axis=0, keepdims=True)
    num_t = jnp.sum(jnp.where(t_iota == idx_a, poss_ref[...], 0),
                    axis=0, keepdims=True)
    mask = t_iota >= num_t

    bits_b = _threefry_bits(_K2B, iota_ref[...])
    aug_ref[...] = jnp.where(mask, 0, (bits_b & _U32(63)).astype(jnp.int32))
    numt_ref[...] = num_t

    ntl = ntl_ref[...]
    m_nt = jnp.max(ntl, axis=0, keepdims=True)
    lse_nt = jnp.log(jnp.sum(jnp.exp(ntl - m_nt), axis=0, keepdims=True)) + m_nt
    lp_nt = ntl - lse_nt
    lp_ref[...] = jnp.sum(jnp.where(t_iota == idx_a, lp_nt, 0.0),
                          axis=0, keepdims=True)


def _scale_kernel(iota_ref, aug_ref, numt_ref, lphead_ref, sl_ref,
                  sc_ref, lp_ref):
    i = pl.program_id(0)

    t_iota = jax.lax.broadcasted_iota(jnp.int32, (R, T), 1)
    mask = t_iota >= numt_ref[...]
    aug = aug_ref[...]

    base_d = jnp.uint32(R * TS) * i.astype(jnp.uint32)
    g_d = _gumbel(_threefry_bits(_K3, base_d + iota_ref[...],
                                 key_prefolded=True))

    sl = sl_ref[...]
    m_tab = jnp.max(sl, axis=1, keepdims=True)
    lse_tab = jnp.log(jnp.sum(jnp.exp(sl - m_tab), axis=1, keepdims=True)) + m_tab

    j_iota = jax.lax.broadcasted_iota(jnp.int32, (R, NUM_TRANSFORMS), 1)
    s_iota = jax.lax.broadcasted_iota(jnp.int32, (R, NUM_SCALES), 1)
    lp_sum = lphead_ref[...]
    chosen_cols = []
    for t in range(T):
        onehot = (j_iota == aug[:, t:t + 1]).astype(jnp.float32)
        gathered = jnp.dot(onehot, sl, preferred_element_type=jnp.float32)
        z_t = g_d[:, t * NUM_SCALES:(t + 1) * NUM_SCALES] + gathered
        max_t = jnp.max(z_t, axis=1, keepdims=True)
        chosen = jnp.min(jnp.where(z_t == max_t, s_iota, NUM_SCALES),
                         axis=1, keepdims=True)
        chosen_cols.append(chosen)
        v_sel = jnp.sum(jnp.where(s_iota == chosen, gathered, 0.0),
                        axis=1, keepdims=True)
        lse_g = jnp.dot(onehot, lse_tab, preferred_element_type=jnp.float32)
        lp_sum = lp_sum + jnp.where(mask[:, t:t + 1], 0.0, v_sel - lse_g)

    sc_ref[...] = jnp.concatenate(chosen_cols, axis=1).reshape(1, R, T)
    lp_ref[...] = lp_sum.reshape(1, R, 1)


def kernel(imgs, num_transforms_logits, scale_logits,
           possible_num_sequential_transforms):
    del imgs
    ntl = num_transforms_logits.reshape(T, 1)
    poss = possible_num_sequential_transforms.reshape(T, 1)

    iota_head = jnp.asarray(np.arange(B, dtype=np.uint32)[None, :] * _U32(T)
                            + np.arange(T, dtype=np.uint32)[:, None])
    iota_d = jnp.asarray(np.arange(R, dtype=np.uint32)[:, None] * _U32(TS)
                         + np.arange(TS, dtype=np.uint32)[None, :]
                         + _U32(_K3[1]))

    aug8, numt, lphead = pl.pallas_call(
        _head_kernel,
        in_specs=[
            pl.BlockSpec((T, B), lambda: (0, 0)),
            pl.BlockSpec((T, 1), lambda: (0, 0)),
            pl.BlockSpec((T, 1), lambda: (0, 0)),
        ],
        out_specs=[
            pl.BlockSpec((T, B), lambda: (0, 0)),
            pl.BlockSpec((1, B), lambda: (0, 0)),
            pl.BlockSpec((1, B), lambda: (0, 0)),
        ],
        out_shape=[
            jax.ShapeDtypeStruct((T, B), jnp.int32),
            jax.ShapeDtypeStruct((1, B), jnp.int32),
            jax.ShapeDtypeStruct((1, B), jnp.float32),
        ],
    )(iota_head, ntl, poss)

    aug = aug8.T
    numt_col = numt.reshape(B, 1)
    lphead_col = lphead.reshape(B, 1)

    sc, lp = pl.pallas_call(
        _scale_kernel,
        grid=(GRID,),
        in_specs=[
            pl.BlockSpec((R, TS), lambda i: (0, 0)),
            pl.BlockSpec((R, T), lambda i: (i, 0)),
            pl.BlockSpec((R, 1), lambda i: (i, 0)),
            pl.BlockSpec((R, 1), lambda i: (i, 0)),
            pl.BlockSpec((NUM_TRANSFORMS, NUM_SCALES), lambda i: (0, 0)),
        ],
        out_specs=[
            pl.BlockSpec((1, R, T), lambda i: (i, 0, 0)),
            pl.BlockSpec((1, R, 1), lambda i: (i, 0, 0)),
        ],
        out_shape=[
            jax.ShapeDtypeStruct((GRID, R, T), jnp.int32),
            jax.ShapeDtypeStruct((GRID, R, 1), jnp.float32),
        ],
    )(iota_d, aug, numt_col, lphead_col, scale_logits)

    return (aug, sc.reshape(B, T), lp.reshape(B))


# speedup vs baseline: 3.2671x; 1.0010x over previous
import numpy as np
import jax
import jax.numpy as jnp
from jax.experimental import pallas as pl
from jax.experimental.pallas import tpu as pltpu

B = 16384
T = 8
NUM_TRANSFORMS = 64
NUM_SCALES = 256

R = 512
GRID = B // R
TS = T * NUM_SCALES

_U32 = np.uint32
_ROT0 = (13, 15, 26, 6)
_ROT1 = (17, 29, 16, 24)


def _threefry_np(k0, k1, x0, x1):
    np.seterr(over="ignore")
    k0, k1 = _U32(k0), _U32(k1)
    ks = [k0, k1, _U32(k0 ^ k1 ^ _U32(0x1BD11BDA))]
    x0 = (x0 + k0).astype(_U32)
    x1 = (x1 + k1).astype(_U32)
    for i in range(5):
        for r in _ROT0 if i % 2 == 0 else _ROT1:
            x0 = (x0 + x1).astype(_U32)
            x1 = ((x1 << _U32(r)) | (x1 >> _U32(32 - r))).astype(_U32)
            x1 = x1 ^ x0
        x0 = (x0 + ks[(i + 1) % 3]).astype(_U32)
        x1 = (x1 + ks[(i + 2) % 3] + _U32(i + 1)).astype(_U32)
    return x0, x1


def _split_np(key, num):
    a, b = _threefry_np(key[0], key[1], np.zeros(num, _U32), np.arange(num, dtype=_U32))
    return list(zip(a.tolist(), b.tolist()))


_K1, _K2, _K3 = _split_np((0, 42), 3)
_K2B = _split_np(_K2, 2)[1]


def _threefry_bits(key, x1, key_prefolded=False):
    ks = (jnp.uint32(key[0]), jnp.uint32(key[1]),
          jnp.uint32(key[0] ^ key[1] ^ 0x1BD11BDA))
    x0 = jnp.full(x1.shape, key[0], jnp.uint32)
    if not key_prefolded:
        x1 = x1 + ks[1]
    for i in range(5):
        for r in _ROT0 if i % 2 == 0 else _ROT1:
            x0 = x0 + x1
            x1 = (x1 << _U32(r)) | (x1 >> _U32(32 - r))
            x1 = x1 ^ x0
        x0 = x0 + ks[(i + 1) % 3]
        x1 = x1 + ks[(i + 2) % 3] + jnp.uint32(i + 1)
    return x0 ^ x1


def _gumbel(bits):
    tiny = np.float32(np.finfo(np.float32).tiny)
    float_bits = (bits >> _U32(9)) | _U32(0x3F800000)
    floats = jax.lax.bitcast_convert_type(float_bits, jnp.float32) - jnp.float32(1.0)
    u = jnp.maximum(tiny, floats * (np.float32(1.0) - tiny) + tiny)
    return -jnp.log(-jnp.log(u))


def _head_kernel(iota_ref, ntl_ref, poss_ref, aug_ref, numt_ref, lp_ref):
    t_iota = jax.lax.broadcasted_iota(jnp.int32, (T, B), 0)

    z_a = _gumbel(_threefry_bits(_K1, iota_ref[...])) + ntl_ref[...]
    max_a = jnp.max(z_a, axis=0, keepdims=True)
    idx_a = jnp.min(jnp.where(z_a == max_a, t_iota, T), axis=0, keepdims=True)
    num_t = jnp.sum(jnp.where(t_iota == idx_a, poss_ref[...], 0),
                    axis=0, keepdims=True)
    mask = t_iota >= num_t

    bits_b = _threefry_bits(_K2B, iota_ref[...])
    aug_ref[...] = jnp.where(mask, 0, (bits_b & _U32(63)).astype(jnp.int32))
    numt_ref[...] = num_t

    ntl = ntl_ref[...]
    m_nt = jnp.max(ntl, axis=0, keepdims=True)
    lse_nt = jnp.log(jnp.sum(jnp.exp(ntl - m_nt), axis=0, keepdims=True)) + m_nt
    lp_nt = ntl - lse_nt
    lp_ref[...] = jnp.sum(jnp.where(t_iota == idx_a, lp_nt, 0.0),
                          axis=0, keepdims=True)


def _scale_kernel(iota_ref, aug_ref, numt_ref, lphead_ref, sl_ref,
                  sc_ref, lp_ref):
    i = pl.program_id(0)

    t_iota = jax.lax.broadcasted_iota(jnp.int32, (R, T), 1)
    mask = t_iota >= numt_ref[...]
    aug = aug_ref[...]

    base_d = jnp.uint32(R * TS) * i.astype(jnp.uint32)
    g_d = _gumbel(_threefry_bits(_K3, base_d + iota_ref[...],
                                 key_prefolded=True))

    sl = sl_ref[...]
    m_tab = jnp.max(sl, axis=1, keepdims=True)
    lse_tab = jnp.log(jnp.sum(jnp.exp(sl - m_tab), axis=1, keepdims=True)) + m_tab

    j_iota = jax.lax.broadcasted_iota(jnp.int32, (R, NUM_TRANSFORMS), 1)
    s_iota = jax.lax.broadcasted_iota(jnp.int32, (R, NUM_SCALES), 1)
    lp_sum = lphead_ref[...]
    chosen_cols = []
    for t in range(T):
        onehot = (j_iota == aug[:, t:t + 1]).astype(jnp.float32)
        gathered = jnp.dot(onehot, sl, preferred_element_type=jnp.float32)
        z_t = g_d[:, t * NUM_SCALES:(t + 1) * NUM_SCALES] + gathered
        max_t = jnp.max(z_t, axis=1, keepdims=True)
        chosen = jnp.min(jnp.where(z_t == max_t, s_iota, NUM_SCALES),
                         axis=1, keepdims=True)
        chosen_cols.append(chosen)
        v_sel = jnp.sum(jnp.where(s_iota == chosen, gathered, 0.0),
                        axis=1, keepdims=True)
        lse_g = jnp.dot(onehot, lse_tab, preferred_element_type=jnp.float32)
        lp_sum = lp_sum + jnp.where(mask[:, t:t + 1], 0.0, v_sel - lse_g)

    sc_ref[...] = jnp.concatenate(chosen_cols, axis=1).reshape(1, R, T)
    lp_ref[...] = lp_sum.reshape(1, R, 1)


def kernel(imgs, num_transforms_logits, scale_logits,
           possible_num_sequential_transforms):
    del imgs
    ntl = num_transforms_logits.reshape(T, 1)
    poss = possible_num_sequential_transforms.reshape(T, 1)

    iota_head = jnp.asarray(np.arange(B, dtype=np.uint32)[None, :] * _U32(T)
                            + np.arange(T, dtype=np.uint32)[:, None])
    iota_d = jnp.asarray(np.arange(R, dtype=np.uint32)[:, None] * _U32(TS)
                         + np.arange(TS, dtype=np.uint32)[None, :]
                         + _U32(_K3[1]))

    aug8, numt, lphead = pl.pallas_call(
        _head_kernel,
        in_specs=[
            pl.BlockSpec((T, B), lambda: (0, 0)),
            pl.BlockSpec((T, 1), lambda: (0, 0)),
            pl.BlockSpec((T, 1), lambda: (0, 0)),
        ],
        out_specs=[
            pl.BlockSpec((T, B), lambda: (0, 0)),
            pl.BlockSpec((1, B), lambda: (0, 0)),
            pl.BlockSpec((1, B), lambda: (0, 0)),
        ],
        out_shape=[
            jax.ShapeDtypeStruct((T, B), jnp.int32),
            jax.ShapeDtypeStruct((1, B), jnp.int32),
            jax.ShapeDtypeStruct((1, B), jnp.float32),
        ],
    )(iota_head, ntl, poss)

    aug = aug8.T
    numt_col = numt.reshape(B, 1)
    lphead_col = lphead.reshape(B, 1)

    sc, lp = pl.pallas_call(
        _scale_kernel,
        grid=(GRID,),
        compiler_params=pltpu.CompilerParams(
            dimension_semantics=("parallel",)),
        in_specs=[
            pl.BlockSpec((R, TS), lambda i: (0, 0)),
            pl.BlockSpec((R, T), lambda i: (i, 0)),
            pl.BlockSpec((R, 1), lambda i: (i, 0)),
            pl.BlockSpec((R, 1), lambda i: (i, 0)),
            pl.BlockSpec((NUM_TRANSFORMS, NUM_SCALES), lambda i: (0, 0)),
        ],
        out_specs=[
            pl.BlockSpec((1, R, T), lambda i: (i, 0, 0)),
            pl.BlockSpec((1, R, 1), lambda i: (i, 0, 0)),
        ],
        out_shape=[
            jax.ShapeDtypeStruct((GRID, R, T), jnp.int32),
            jax.ShapeDtypeStruct((GRID, R, 1), jnp.float32),
        ],
    )(iota_d, aug, numt_col, lphead_col, scale_logits)

    return (aug, sc.reshape(B, T), lp.reshape(B))
